# Initial kernel scaffold; baseline (speedup 1.0000x reference)
import jax
import jax.numpy as jnp
from jax.experimental import pallas as pl


def kernel(x_n, A_n, A_s, A_n_ts, A_n_cs, x_p, A_p, W1_1, b1_1, W1_2, b1_2, W1_3, b1_3, W1_4, b1_4, W2_1, b2_1, W2_2, b2_2, W2_3, b2_3, W2_4, b2_4, W2_5, b2_5, W3_1, b3_1, W3_2, b3_2, W4_1, b4_1, W4_2, b4_2):
    raise NotImplementedError("write your pallas kernel here")


# speedup vs baseline: 1.3068x; 1.3068x over previous
import jax
import jax.numpy as jnp
from jax import lax
from jax.experimental import pallas as pl
from jax.experimental.pallas import tpu as pltpu

_N, _NP, _F, _H = 4096, 1024, 64, 32
_BM = 256
_NBLK = _N // _BM
_F32 = jnp.float32


def _dot(a, b):
    return jnp.dot(a, b, preferred_element_type=_F32)


def _relu(x):
    return jnp.maximum(x, 0.0)


def _l1_body(xn, xp, ap, w1, b1, w14, b14, an, x1_out, pooled_out, p_ref):
    i = pl.program_id(0)

    @pl.when(i == 0)
    def _():
        p_ref[...] = _dot(xn[...], w1[...])
        p4 = _dot(xp[...], w14[...])
        x14 = _relu(_dot(ap[...], p4) + b14[...])
        ones = jnp.ones((_NP, 1), dtype=_F32)
        pooled_out[...] = lax.dot_general(
            x14, ones, (((0,), (0,)), ((), ())), preferred_element_type=_F32
        )

    x1_out[...] = _relu(_dot(an[...], p_ref[...]) + b1[...])


def _l2_body(x1, pooled, w2, b2, w25, an, ats, acs, as_, s_out,
             p21_ref, p25_ref, p22_ref, p23_ref, p24_ref):
    i = pl.program_id(0)

    @pl.when(i == 0)
    def _():
        x11 = x1[:, 0:_H]
        x12 = x1[:, _H:2 * _H]
        x13 = x1[:, 2 * _H:3 * _H]
        p21_ref[...] = _dot(x11, w2[:, 0:_H])
        p22_ref[...] = _dot(x12, w2[:, _H:2 * _H])
        p23_ref[...] = _dot(x12, w2[:, 2 * _H:3 * _H])
        p24_ref[...] = _dot(x13, w2[:, 3 * _H:4 * _H])
        wsum = jnp.sum(w25[...], axis=0, keepdims=True)
        outer = _dot(pooled[...], wsum)
        r = lax.broadcasted_iota(jnp.int32, (_N, _H), 0) // 128
        c = lax.broadcasted_iota(jnp.int32, (_N, _H), 1)
        m = (r == c).astype(_F32)
        p25_ref[...] = _dot(m, outer)

    a_n = an[...]
    s = (_relu(_dot(a_n, p21_ref[...]) + b2[:, 0:_H])
         + _relu(_dot(a_n, p25_ref[...]) + b2[:, 4 * _H:5 * _H])
         + _relu(_dot(ats[...], p22_ref[...]) + b2[:, _H:2 * _H])
         + _relu(_dot(acs[...], p23_ref[...]) + b2[:, 2 * _H:3 * _H])
         + _relu(_dot(as_[...], p24_ref[...]) + b2[:, 3 * _H:4 * _H]))
    s_out[...] = s


def _l3_body(s_in, w3, b3, an, as_, x3_out, p_ref):
    i = pl.program_id(0)

    @pl.when(i == 0)
    def _():
        p_ref[...] = _dot(s_in[...], w3[...])

    t1 = _relu(_dot(an[...], p_ref[:, 0:_H]) + b3[:, 0:_H])
    t2 = _relu(_dot(as_[...], p_ref[:, _H:2 * _H]) + b3[:, _H:2 * _H])
    x3_out[...] = jnp.concatenate([t1, t2], axis=1)


def _l4_body(x3, w41, b41, w42, b42, an, o1_out, o2_out, p41_ref, p42_ref):
    i = pl.program_id(0)

    @pl.when(i == 0)
    def _():
        p41_ref[...] = _dot(x3[:, 0:_H], w41[...])
        p42_ref[...] = _dot(x3[:, _H:2 * _H], w42[...])

    a_n = an[...]
    o1_out[...] = jax.nn.sigmoid(_dot(a_n, p41_ref[...]) + b41[...])
    o2_out[...] = jax.nn.sigmoid(_dot(a_n, p42_ref[...]) + b42[...])


def _full(shape):
    return pl.BlockSpec(shape, lambda i: (0,) * len(shape))


def _rows(width):
    return pl.BlockSpec((_BM, width), lambda i: (i, 0))


_PARAMS = pltpu.CompilerParams(
    dimension_semantics=("arbitrary",),
    vmem_limit_bytes=100 * 1024 * 1024,
)


def kernel(x_n, A_n, A_s, A_n_ts, A_n_cs, x_p, A_p,
           W1_1, b1_1, W1_2, b1_2, W1_3, b1_3, W1_4, b1_4,
           W2_1, b2_1, W2_2, b2_2, W2_3, b2_3, W2_4, b2_4, W2_5, b2_5,
           W3_1, b3_1, W3_2, b3_2, W4_1, b4_1, W4_2, b4_2):
    xn = x_n[0]
    xp = x_p[0]
    an = A_n[0]
    as_ = A_s[0]
    ats = A_n_ts[0]
    acs = A_n_cs[0]
    ap = A_p[0]

    w1 = jnp.concatenate([W1_1, W1_2, W1_3], axis=1)
    b1 = jnp.concatenate([b1_1, b1_2, b1_3])[None, :]
    w2 = jnp.concatenate([W2_1, W2_2, W2_3, W2_4], axis=1)
    b2 = jnp.concatenate([b2_1, b2_2, b2_3, b2_4, b2_5])[None, :]
    w3 = jnp.concatenate([W3_1, W3_2], axis=1)
    b3 = jnp.concatenate([b3_1, b3_2])[None, :]

    x1, pooled = pl.pallas_call(
        _l1_body,
        grid=(_NBLK,),
        in_specs=[
            _full((_N, _F)), _full((_NP, _F)), _full((_NP, _NP)),
            _full((_F, 3 * _H)), _full((1, 3 * _H)),
            _full((_F, _H)), _full((1, _H)),
            _rows(_N),
        ],
        out_specs=[_rows(3 * _H), _full((_H, 1))],
        out_shape=[
            jax.ShapeDtypeStruct((_N, 3 * _H), _F32),
            jax.ShapeDtypeStruct((_H, 1), _F32),
        ],
        scratch_shapes=[pltpu.VMEM((_N, 3 * _H), _F32)],
        compiler_params=_PARAMS,
    )(xn, xp, ap, w1, b1, W1_4, b1_4[None, :], an)

    s = pl.pallas_call(
        _l2_body,
        grid=(_NBLK,),
        in_specs=[
            _full((_N, 3 * _H)), _full((_H, 1)),
            _full((_H, 4 * _H)), _full((1, 5 * _H)), _full((_H, _H)),
            _rows(_N), _rows(_N), _rows(_N), _rows(_N),
        ],
        out_specs=_rows(_H),
        out_shape=jax.ShapeDtypeStruct((_N, _H), _F32),
        scratch_shapes=[pltpu.VMEM((_N, _H), _F32)] * 5,
        compiler_params=_PARAMS,
    )(x1, pooled, w2, b2, W2_5, an, ats, acs, as_)

    x3 = pl.pallas_call(
        _l3_body,
        grid=(_NBLK,),
        in_specs=[
            _full((_N, _H)), _full((_H, 2 * _H)), _full((1, 2 * _H)),
            _rows(_N), _rows(_N),
        ],
        out_specs=_rows(2 * _H),
        out_shape=jax.ShapeDtypeStruct((_N, 2 * _H), _F32),
        scratch_shapes=[pltpu.VMEM((_N, 2 * _H), _F32)],
        compiler_params=_PARAMS,
    )(s, w3, b3, an, as_)

    a1 = W4_1.shape[1]
    a2 = W4_2.shape[1]
    out1, out2 = pl.pallas_call(
        _l4_body,
        grid=(_NBLK,),
        in_specs=[
            _full((_N, 2 * _H)),
            _full((_H, a1)), _full((1, a1)),
            _full((_H, a2)), _full((1, a2)),
            _rows(_N),
        ],
        out_specs=[_rows(a1), _rows(a2)],
        out_shape=[
            jax.ShapeDtypeStruct((_N, a1), _F32),
            jax.ShapeDtypeStruct((_N, a2), _F32),
        ],
        scratch_shapes=[
            pltpu.VMEM((_N, a1), _F32),
            pltpu.VMEM((_N, a2), _F32),
        ],
        compiler_params=_PARAMS,
    )(x3, W4_1, b4_1[None, :], W4_2, b4_2[None, :], an)

    return (out1[None], out2[None])
